# Initial kernel scaffold; baseline (speedup 1.0000x reference)
import jax
import jax.numpy as jnp
from jax.experimental import pallas as pl


def kernel(f_src, f_dst, h0, x, h_att, lg_edge_index, segment_ids, wz, uz, bz, wr, ur, br, w, u, b, a):
    raise NotImplementedError("write your pallas kernel here")


# speedup vs baseline: 1.0635x; 1.0635x over previous
import functools

import jax
import jax.numpy as jnp
from jax import lax
from jax.experimental import pallas as pl
from jax.experimental.pallas import tpu as pltpu

E = 160000
E_LG = 320000
N = 10000
B = 100
D = 256

BM = 1000


def _phase_a_body(f_dst_ref, h0_ref, wr_ref, ur_ref, br_ref, out_ref):
    f_dst = f_dst_ref[...]
    h0 = h0_ref[...]
    acc = jnp.dot(f_dst.astype(jnp.bfloat16), wr_ref[...].astype(jnp.bfloat16),
                  preferred_element_type=jnp.float32)
    acc += jnp.dot(h0.astype(jnp.bfloat16), ur_ref[...].astype(jnp.bfloat16),
                   preferred_element_type=jnp.float32)
    r = jax.nn.sigmoid(acc + br_ref[...])
    out_ref[...] = r * h0


def _phase_a(f_dst, h0, wr, ur, br):
    grid = (E // BM,)
    return pl.pallas_call(
        _phase_a_body,
        grid=grid,
        in_specs=[
            pl.BlockSpec((BM, D), lambda i: (i, 0)),
            pl.BlockSpec((BM, D), lambda i: (i, 0)),
            pl.BlockSpec((D, D), lambda i: (0, 0)),
            pl.BlockSpec((D, D), lambda i: (0, 0)),
            pl.BlockSpec((1, D), lambda i: (0, 0)),
        ],
        out_specs=pl.BlockSpec((BM, D), lambda i: (i, 0)),
        out_shape=jax.ShapeDtypeStruct((E, D), jnp.float32),
    )(f_dst, h0, wr, ur, br)


def _phase_c_body(f_src_ref, s_ref, srh_ref, wz_ref, uz_ref, bz_ref,
                  w_ref, u_ref, b_ref, out_ref):
    f_src = f_src_ref[...].astype(jnp.bfloat16)
    s = s_ref[...]
    pre_z = jnp.dot(f_src, wz_ref[...].astype(jnp.bfloat16),
                    preferred_element_type=jnp.float32)
    pre_z += jnp.dot(s.astype(jnp.bfloat16), uz_ref[...].astype(jnp.bfloat16),
                     preferred_element_type=jnp.float32)
    z = jax.nn.sigmoid(pre_z + bz_ref[...])
    pre_h = jnp.dot(f_src, w_ref[...].astype(jnp.bfloat16),
                    preferred_element_type=jnp.float32)
    pre_h += jnp.dot(srh_ref[...].astype(jnp.bfloat16),
                     u_ref[...].astype(jnp.bfloat16),
                     preferred_element_type=jnp.float32)
    h_tilde = jnp.tanh(pre_h + b_ref[...])
    out_ref[...] = (1.0 - z) * s + z * h_tilde


def _phase_c(f_src, s, srh, wz, uz, bz, w, u, b):
    grid = (E // BM,)
    return pl.pallas_call(
        _phase_c_body,
        grid=grid,
        in_specs=[
            pl.BlockSpec((BM, D), lambda i: (i, 0)),
            pl.BlockSpec((BM, D), lambda i: (i, 0)),
            pl.BlockSpec((BM, D), lambda i: (i, 0)),
            pl.BlockSpec((D, D), lambda i: (0, 0)),
            pl.BlockSpec((D, D), lambda i: (0, 0)),
            pl.BlockSpec((1, D), lambda i: (0, 0)),
            pl.BlockSpec((D, D), lambda i: (0, 0)),
            pl.BlockSpec((D, D), lambda i: (0, 0)),
            pl.BlockSpec((1, D), lambda i: (0, 0)),
        ],
        out_specs=pl.BlockSpec((BM, D), lambda i: (i, 0)),
        out_shape=jax.ShapeDtypeStruct((E, D), jnp.float32),
    )(f_src, s, srh, wz, uz, bz, w, u, b)


def _attention_body(x_ref, h_att_ref, a_ref, seg_ref, out_ref):
    seg = seg_ref[0, :]
    onehot = (seg[:, None] ==
              lax.broadcasted_iota(jnp.int32, (N, B), 1)).astype(jnp.float32)
    proj = jnp.dot(h_att_ref[...], a_ref[...],
                   preferred_element_type=jnp.float32)
    x = x_ref[...]
    projseg = jnp.dot(onehot, proj, preferred_element_type=jnp.float32)
    s_att = jnp.sum(x * projseg, axis=1)
    neg = jnp.float32(-1e30)
    masked = jnp.where(onehot > 0.5, s_att[:, None], neg)
    smax = jnp.max(masked, axis=0)
    e = jnp.exp(s_att - jnp.dot(onehot, smax))
    zsum = jnp.sum(onehot * e[:, None], axis=0)
    attw = e / jnp.dot(onehot, zsum)
    ret = lax.dot_general(onehot * attw[:, None], x,
                          dimension_numbers=(((0,), (0,)), ((), ())),
                          preferred_element_type=jnp.float32)
    out_ref[...] = ret


def _attention(x, h_att, a, seg_i32):
    seg2d = seg_i32.reshape(1, N)
    return pl.pallas_call(
        _attention_body,
        in_specs=[
            pl.BlockSpec((N, D), lambda: (0, 0)),
            pl.BlockSpec((B, D), lambda: (0, 0)),
            pl.BlockSpec((D, D), lambda: (0, 0)),
            pl.BlockSpec((1, N), lambda: (0, 0)),
        ],
        out_specs=pl.BlockSpec((B, D), lambda: (0, 0)),
        out_shape=jax.ShapeDtypeStruct((B, D), jnp.float32),
    )(x, h_att, a, seg2d)


def kernel(f_src, f_dst, h0, x, h_att, lg_edge_index, segment_ids,
           wz, uz, bz, wr, ur, br, w, u, b, a):
    src = lg_edge_index[0].astype(jnp.int32)
    dst = lg_edge_index[1].astype(jnp.int32)
    seg_i32 = segment_ids.astype(jnp.int32)

    rh = _phase_a(f_dst, h0, wr, ur, br)

    s = jax.ops.segment_sum(h0[src], dst, num_segments=E)
    srh = jax.ops.segment_sum(rh[src], dst, num_segments=E)

    h = _phase_c(f_src, s, srh, wz, uz, bz, w, u, b)
    ret = _attention(x, h_att, a, seg_i32)
    return h, ret
